# Initial kernel scaffold; baseline (speedup 1.0000x reference)
import jax
import jax.numpy as jnp
from jax.experimental import pallas as pl


def kernel(input, target, smooth):
    raise NotImplementedError("write your pallas kernel here")


# speedup vs baseline: 5.1449x; 5.1449x over previous
import jax
import jax.numpy as jnp
from jax.experimental import pallas as pl
from jax.experimental.pallas import tpu as pltpu

_B, _C, _H, _W = 8, 17, 512, 512
_ROWS = 64
_GRID = (_B, _H // _ROWS)
_NBLK = _GRID[0] * _GRID[1]


def _dice_body(smooth_ref, x_ref, t_ref, out_ref, acc_ref):
    b = pl.program_id(0)
    i = pl.program_id(1)
    pid = b * _GRID[1] + i

    @pl.when(pid == 0)
    def _init():
        acc_ref[...] = jnp.zeros_like(acc_ref)

    x = x_ref[0]
    t = t_ref[0]
    s = 1.0 / (1.0 + jnp.exp(-x))
    cls = jax.lax.broadcasted_iota(jnp.int32, (_C, _ROWS, _W), 0)
    mf = (cls == t[None]).astype(jnp.float32)

    acc_ref[0] += jnp.sum(s, axis=1)
    acc_ref[1] += jnp.sum(s * mf, axis=1)
    acc_ref[2] += jnp.sum(mf, axis=1)

    @pl.when(pid == _NBLK - 1)
    def _finish():
        smooth = smooth_ref[0, 0]
        sumx = jnp.sum(acc_ref[0], axis=1)
        inter = jnp.sum(acc_ref[1])
        cnt = jnp.sum(acc_ref[2], axis=1)
        denom = sumx + cnt
        out_ref[0, 0] = -(2.0 * inter + jnp.sum(smooth / (denom + smooth)))


def kernel(input, target, smooth):
    smooth2d = jnp.reshape(smooth, (1, 1)).astype(jnp.float32)
    out = pl.pallas_call(
        _dice_body,
        grid=_GRID,
        in_specs=[
            pl.BlockSpec(memory_space=pltpu.SMEM),
            pl.BlockSpec((1, _C, _ROWS, _W), lambda b, i: (b, 0, i, 0)),
            pl.BlockSpec((1, _ROWS, _W), lambda b, i: (b, i, 0)),
        ],
        out_specs=pl.BlockSpec(memory_space=pltpu.SMEM),
        out_shape=jax.ShapeDtypeStruct((1, 1), jnp.float32),
        scratch_shapes=[pltpu.VMEM((3, _C, _W), jnp.float32)],
    )(smooth2d, input, target)
    return out[0, 0]


# speedup vs baseline: 5.1712x; 1.0051x over previous
import jax
import jax.numpy as jnp
from jax.experimental import pallas as pl
from jax.experimental.pallas import tpu as pltpu

_B, _C, _H, _W = 8, 17, 512, 512
_ROWS = 64
_GRID = (_B, _H // _ROWS)
_NBLK = _GRID[0] * _GRID[1]


def _dice_body(smooth_ref, x_ref, t_ref, out_ref, acc_ref):
    b = pl.program_id(0)
    i = pl.program_id(1)
    pid = b * _GRID[1] + i

    @pl.when(pid == 0)
    def _init():
        acc_ref[...] = jnp.zeros_like(acc_ref)

    x = x_ref[0]
    t = t_ref[0]
    s = 0.5 * jnp.tanh(0.5 * x) + 0.5
    cls = jax.lax.broadcasted_iota(jnp.int32, (_C, _ROWS, _W), 0)
    m = cls == t[None]

    acc_ref[0] += jnp.sum(jnp.where(m, s + 1.0, s), axis=1)
    acc_ref[1] += jnp.sum(jnp.where(m, s, 0.0), axis=1)

    @pl.when(pid == _NBLK - 1)
    def _finish():
        smooth = smooth_ref[0, 0]
        denom = jnp.sum(acc_ref[0], axis=1)
        inter = jnp.sum(acc_ref[1])
        out_ref[0, 0] = -(2.0 * inter + jnp.sum(smooth / (denom + smooth)))


def kernel(input, target, smooth):
    smooth2d = jnp.reshape(smooth, (1, 1)).astype(jnp.float32)
    out = pl.pallas_call(
        _dice_body,
        grid=_GRID,
        in_specs=[
            pl.BlockSpec(memory_space=pltpu.SMEM),
            pl.BlockSpec((1, _C, _ROWS, _W), lambda b, i: (b, 0, i, 0)),
            pl.BlockSpec((1, _ROWS, _W), lambda b, i: (b, i, 0)),
        ],
        out_specs=pl.BlockSpec(memory_space=pltpu.SMEM),
        out_shape=jax.ShapeDtypeStruct((1, 1), jnp.float32),
        scratch_shapes=[pltpu.VMEM((2, _C, _W), jnp.float32)],
    )(smooth2d, input, target)
    return out[0, 0]


# speedup vs baseline: 6.5966x; 1.2756x over previous
import jax
import jax.numpy as jnp
from jax.experimental import pallas as pl
from jax.experimental.pallas import tpu as pltpu

_B, _C, _H, _W = 8, 17, 512, 512
_ROWS = 128
_GRID = (_B, _H // _ROWS)
_NBLK = _GRID[0] * _GRID[1]
_NPIX = float(_B * _H * _W)


def _dice_body(smooth_ref, x_ref, t_ref, out_ref, acc_ref):
    b = pl.program_id(0)
    i = pl.program_id(1)
    pid = b * _GRID[1] + i

    @pl.when(pid == 0)
    def _init():
        acc_ref[...] = jnp.zeros_like(acc_ref)

    x = x_ref[0]
    t = t_ref[0]
    T = jnp.tanh(0.5 * x)
    cls = jax.lax.broadcasted_iota(jnp.int32, (_C, _ROWS, _W), 0)
    m = cls == t[None]

    acc_ref[0] += jnp.sum(jnp.where(m, T + 2.0, T), axis=1)
    acc_ref[1] += jnp.sum(jnp.where(m, T, 0.0), axis=1)

    @pl.when(pid == _NBLK - 1)
    def _finish():
        smooth = smooth_ref[0, 0]
        denom = 0.5 * jnp.sum(acc_ref[0], axis=1) + (0.5 * _NPIX)
        inter = 0.5 * jnp.sum(acc_ref[1]) + (0.5 * _NPIX)
        out_ref[0, 0] = -(2.0 * inter + jnp.sum(smooth / (denom + smooth)))


def kernel(input, target, smooth):
    smooth2d = jnp.reshape(smooth, (1, 1)).astype(jnp.float32)
    out = pl.pallas_call(
        _dice_body,
        grid=_GRID,
        in_specs=[
            pl.BlockSpec(memory_space=pltpu.SMEM),
            pl.BlockSpec((1, _C, _ROWS, _W), lambda b, i: (b, 0, i, 0)),
            pl.BlockSpec((1, _ROWS, _W), lambda b, i: (b, i, 0)),
        ],
        out_specs=pl.BlockSpec(memory_space=pltpu.SMEM),
        out_shape=jax.ShapeDtypeStruct((1, 1), jnp.float32),
        scratch_shapes=[pltpu.VMEM((2, _C, _W), jnp.float32)],
    )(smooth2d, input, target)
    return out[0, 0]
